# Initial kernel scaffold; baseline (speedup 1.0000x reference)
import jax
import jax.numpy as jnp
from jax.experimental import pallas as pl


def kernel(hidden_states, router_weight, w_gate_up, w_down):
    raise NotImplementedError("write your pallas kernel here")


# speedup vs baseline: 1.2520x; 1.2520x over previous
import jax
import jax.numpy as jnp
from jax.experimental import pallas as pl
from jax.experimental.pallas import tpu as pltpu

T = 1024
D = 1024
E = 8
TOPK = 2
FF = 512
TT = 256
NT = T // TT


def _moe_dense_kernel(logits_ref, x_ref, wgu_ref, wd_ref, out_ref,
                      wgu_bf, wd_bf):
    e = pl.program_id(0)
    t = pl.program_id(1)

    @pl.when(t == 0)
    def _():
        wgu_bf[...] = wgu_ref[0].astype(jnp.bfloat16)
        wd_bf[...] = wd_ref[0].astype(jnp.bfloat16)

    tsl = pl.ds(t * TT, TT)

    lg = logits_ref[tsl, :]
    m = jnp.max(lg, axis=-1, keepdims=True)
    ex = jnp.exp(lg - m)
    p = ex / jnp.sum(ex, axis=-1, keepdims=True)
    lane = jax.lax.broadcasted_iota(jnp.int32, (TT, E), 1)
    v1 = jnp.max(p, axis=-1, keepdims=True)
    c1 = jnp.min(jnp.where(p == v1, lane, E), axis=-1, keepdims=True)
    m1 = lane == c1
    p2 = jnp.where(m1, -1.0, p)
    v2 = jnp.max(p2, axis=-1, keepdims=True)
    c2 = jnp.min(jnp.where(p2 == v2, lane, E), axis=-1, keepdims=True)
    m2 = lane == c2
    denom = v1 + v2
    comb = (jnp.where(m1, v1, 0.0) + jnp.where(m2, v2, 0.0)) / denom
    scale = jnp.sum(jnp.where(lane == e, comb, 0.0), axis=-1,
                    keepdims=True)

    xt = x_ref[tsl, :].astype(jnp.bfloat16)
    gu = jax.lax.dot_general(
        xt, wgu_bf[...], (((1,), (1,)), ((), ())),
        preferred_element_type=jnp.float32)
    g = gu[:, :FF]
    u = gu[:, FF:]
    act = (g * jax.nn.sigmoid(g)) * u
    act = (act * scale).astype(jnp.bfloat16)
    y = jax.lax.dot_general(
        act, wd_bf[...], (((1,), (1,)), ((), ())),
        preferred_element_type=jnp.float32)

    @pl.when(e == 0)
    def _():
        out_ref[tsl, :] = y

    @pl.when(e != 0)
    def _():
        out_ref[tsl, :] = out_ref[tsl, :] + y


def kernel(hidden_states, router_weight, w_gate_up, w_down):
    router_logits = hidden_states @ router_weight.T

    out = pl.pallas_call(
        _moe_dense_kernel,
        grid=(E, NT),
        in_specs=[
            pl.BlockSpec((T, E), lambda e, t: (0, 0)),
            pl.BlockSpec((T, D), lambda e, t: (0, 0)),
            pl.BlockSpec((1, 2 * FF, D), lambda e, t: (e, 0, 0)),
            pl.BlockSpec((1, D, FF), lambda e, t: (e, 0, 0)),
        ],
        out_specs=pl.BlockSpec((T, D), lambda e, t: (0, 0)),
        out_shape=jax.ShapeDtypeStruct((T, D), jnp.float32),
        scratch_shapes=[
            pltpu.VMEM((2 * FF, D), jnp.bfloat16),
            pltpu.VMEM((D, FF), jnp.bfloat16),
        ],
    )(router_logits, hidden_states, w_gate_up, w_down)
    return out
